# Initial kernel scaffold; baseline (speedup 1.0000x reference)
import jax
import jax.numpy as jnp
from jax.experimental import pallas as pl


def kernel(x, table):
    raise NotImplementedError("write your pallas kernel here")


# speedup vs baseline: 4.6202x; 4.6202x over previous
import functools

import jax
import jax.numpy as jnp
from jax import lax
from jax.experimental import pallas as pl
from jax.experimental.pallas import tpu as pltpu
from jax.experimental.pallas import tpu_sc as plsc

NUM_EMB = 100000
DIM = 64
NB = 4096
NS_SEQ = 50

_info = plsc.get_sparse_core_info()
NC = _info.num_cores
NSUB = _info.num_subcores
NW = NC * NSUB
ROWS_PER_W = NB // NW
ROWS_PER_CHUNK = 16
NCHUNK = ROWS_PER_W // ROWS_PER_CHUNK

_mesh = plsc.VectorSubcoreMesh(core_axis_name="c", subcore_axis_name="s")


@functools.partial(
    pl.kernel,
    mesh=_mesh,
    out_type=jax.ShapeDtypeStruct((NB, NS_SEQ, DIM), jnp.float32),
    compiler_params=pltpu.CompilerParams(use_tc_tiling_on_sc=False),
    scratch_types=[
        pltpu.VMEM((ROWS_PER_W, NS_SEQ), jnp.int32),
        pltpu.VMEM((2, ROWS_PER_CHUNK, NS_SEQ, DIM), jnp.float32),
        pltpu.SemaphoreType.DMA,
        pltpu.SemaphoreType.DMA,
        pltpu.SemaphoreType.DMA,
        pltpu.SemaphoreType.DMA,
    ],
)
def _gather_kernel(table_hbm, x_hbm, out_hbm, idx_v, rows_v, g0, g1, o0, o1):
    wid = lax.axis_index("s") * NC + lax.axis_index("c")
    xrow0 = wid * ROWS_PER_W
    gsem = (g0, g1)
    osem = (o0, o1)
    pltpu.sync_copy(x_hbm.at[pl.ds(xrow0, ROWS_PER_W)], idx_v)

    def gather(i):
        b = i % 2
        cs = []
        for j in range(ROWS_PER_CHUNK):
            cs.append(
                pltpu.async_copy(
                    table_hbm.at[idx_v.at[i * ROWS_PER_CHUNK + j]],
                    rows_v.at[b, j],
                    gsem[b],
                )
            )
        return cs

    def put(i):
        return pltpu.async_copy(
            rows_v.at[i % 2],
            out_hbm.at[pl.ds(xrow0 + i * ROWS_PER_CHUNK, ROWS_PER_CHUNK)],
            osem[i % 2],
        )

    gathers = [None] * NCHUNK
    puts = [None] * NCHUNK
    gathers[0] = gather(0)
    for i in range(NCHUNK):
        if i + 1 < NCHUNK:
            if i >= 1:
                puts[i - 1].wait()
            gathers[i + 1] = gather(i + 1)
        for c in gathers[i]:
            c.wait()
        puts[i] = put(i)
    puts[NCHUNK - 2].wait()
    puts[NCHUNK - 1].wait()


def kernel(x, table):
    return _gather_kernel(table, x.astype(jnp.int32))
